# baseline (speedup 1.0000x reference)
import functools

import jax
import jax.numpy as jnp
from jax import lax
from jax.experimental import pallas as pl
from jax.experimental.pallas import tpu as pltpu
from jax.experimental.pallas import tpu_sc as plsc

N_POINTS = 8192
K_NBRS = 8
KNN_ROWS = 256
NODE_TILE = 512
PAD_F = 128

SC_CORES = 2
SC_SUBCORES = 16
SC_WORKERS = SC_CORES * SC_SUBCORES


_SORT8 = [(0, 1), (2, 3), (4, 5), (6, 7),
          (0, 2), (1, 3), (4, 6), (5, 7),
          (1, 2), (5, 6),
          (0, 4), (1, 5), (2, 6), (3, 7),
          (2, 4), (3, 5),
          (1, 2), (3, 4), (5, 6)]
_BITONIC8 = [(0, 4), (1, 5), (2, 6), (3, 7),
             (0, 2), (1, 3), (4, 6), (5, 7),
             (0, 1), (2, 3), (4, 5), (6, 7)]


def _knn_body(xr_ref, xt_ref, out_ref):
    xr = xr_ref[...]
    xt = xt_ref[...]
    colsq = jnp.sum(xt * xt, axis=0, keepdims=True)
    cross = jax.lax.dot_general(
        xr, xt, (((1,), (0,)), ((), ())),
        preferred_element_type=jnp.float32)
    dist = colsq - 2.0 * cross

    bits = lax.bitcast_convert_type(dist, jnp.int32)
    sortable = bits ^ (lax.shift_right_arithmetic(bits, 31)
                       & jnp.int32(2 ** 31 - 1))
    keys0 = sortable & jnp.int32(-512)

    w = N_POINTS // K_NBRS
    vals = [keys0[:, s * w:(s + 1) * w] | jnp.int32(s * 64)
            for s in range(K_NBRS)]

    def ce(i, j):
        vi, vj = vals[i], vals[j]
        vals[i] = jnp.minimum(vi, vj)
        vals[j] = jnp.maximum(vi, vj)

    for i, j in _SORT8:
        ce(i, j)
    while w > 16:
        w //= 2
        vals = [jnp.minimum(vals[i][:, :w],
                            vals[7 - i][:, w:] + jnp.int32(w // 16))
                for i in range(K_NBRS)]
        for i, j in _BITONIC8:
            ce(i, j)

    ck = jnp.concatenate(vals, axis=1)
    rows = ck.shape[0]
    pos16 = lax.broadcasted_iota(jnp.int32, (rows, 128), 1) & jnp.int32(15)
    big_i = jnp.int32(2 ** 31 - 1)
    for k in range(K_NBRS):
        m = jnp.min(ck, axis=1, keepdims=True)
        p = jnp.min(jnp.where(ck == m, pos16, jnp.int32(16)), axis=1)
        col = (m[:, 0] & jnp.int32(511)) * 16 + p
        out_ref[0, k, :] = col
        ck = jnp.where(ck == m, big_i, ck)


def _knn_indices(xpad, xt):
    grid = N_POINTS // KNN_ROWS
    out = pl.pallas_call(
        _knn_body,
        grid=(grid,),
        in_specs=[
            pl.BlockSpec((KNN_ROWS, PAD_F), lambda i: (i, 0)),
            pl.BlockSpec((PAD_F, N_POINTS), lambda i: (0, 0)),
        ],
        out_specs=pl.BlockSpec((1, K_NBRS, KNN_ROWS), lambda i: (i, 0, 0)),
        out_shape=jax.ShapeDtypeStruct((grid, K_NBRS, KNN_ROWS), jnp.int32),
    )(xpad, xt)
    return jnp.transpose(out, (0, 2, 1)).reshape(N_POINTS, K_NBRS)


def _make_sc_gather(n_idx, feat, chunk):
    per_worker = n_idx // SC_WORKERS
    n_chunks = per_worker // chunk
    mesh = plsc.VectorSubcoreMesh(core_axis_name="c", subcore_axis_name="s")

    @functools.partial(
        pl.kernel,
        out_type=jax.ShapeDtypeStruct((n_idx, feat), jnp.float32),
        mesh=mesh,
        scratch_types=[
            pltpu.VMEM((chunk,), jnp.int32),
            pltpu.VMEM((chunk, feat), jnp.float32),
            pltpu.SemaphoreType.DMA,
        ],
    )
    def gather(table_hbm, idx_hbm, out_hbm, idx_v, rows_v, sem):
        wid = lax.axis_index("s") * SC_CORES + lax.axis_index("c")
        for c in range(n_chunks):
            base = wid * per_worker + c * chunk
            pltpu.sync_copy(idx_hbm.at[pl.ds(base, chunk)], idx_v)
            pltpu.async_copy(table_hbm.at[idx_v], rows_v, sem).wait()
            pltpu.sync_copy(rows_v, out_hbm.at[pl.ds(base, chunk)])

    return gather


def _mlp_body(x_ref, g_ref, wt_ref, wb_ref, ba_ref, w2_ref, b2_ref, o_ref,
              *, relu_out):
    xi = x_ref[...]
    t_nodes, feat = xi.shape
    n_edges = t_nodes * K_NBRS
    xj = g_ref[...].reshape(t_nodes, K_NBRS, feat)
    d = (xj - xi[:, None, :]).reshape(n_edges, feat)
    t1 = jax.lax.dot_general(xi, wt_ref[...], (((1,), (0,)), ((), ())),
                             preferred_element_type=jnp.float32)
    t2 = jax.lax.dot_general(d, wb_ref[...], (((1,), (0,)), ((), ())),
                             preferred_element_type=jnp.float32)
    hdim = t1.shape[1]
    pre = (t1[:, None, :] + t2.reshape(t_nodes, K_NBRS, hdim)
           + ba_ref[...][None])
    h = jnp.maximum(pre, 0.0).reshape(n_edges, hdim)
    h2 = jax.lax.dot_general(h, w2_ref[...], (((1,), (0,)), ((), ())),
                             preferred_element_type=jnp.float32) + b2_ref[...]
    odim = h2.shape[1]
    out = jnp.max(h2.reshape(t_nodes, K_NBRS, odim), axis=1)
    if relu_out:
        out = jnp.maximum(out, 0.0)
    o_ref[...] = out


def _edge_mlp(x, gathered, wt, wb, ba, w2, b2, relu_out):
    t = NODE_TILE
    grid = N_POINTS // t
    feat = x.shape[1]
    hdim = wt.shape[1]
    odim = w2.shape[1]
    body = functools.partial(_mlp_body, relu_out=relu_out)
    return pl.pallas_call(
        body,
        grid=(grid,),
        in_specs=[
            pl.BlockSpec((t, feat), lambda i: (i, 0)),
            pl.BlockSpec((t * K_NBRS, feat), lambda i: (i, 0)),
            pl.BlockSpec((feat, hdim), lambda i: (0, 0)),
            pl.BlockSpec((feat, hdim), lambda i: (0, 0)),
            pl.BlockSpec((1, hdim), lambda i: (0, 0)),
            pl.BlockSpec((hdim, odim), lambda i: (0, 0)),
            pl.BlockSpec((1, odim), lambda i: (0, 0)),
        ],
        out_specs=pl.BlockSpec((t, odim), lambda i: (i, 0)),
        out_shape=jax.ShapeDtypeStruct((N_POINTS, odim), jnp.float32),
    )(x, gathered, wt, wb, ba, w2, b2)


def kernel(point_cloud, W1a, b1a, W1b, b1b, W2a, b2a, W2b, b2b):
    batch, pts, coords = point_cloud.shape
    xf = point_cloud.reshape(-1, coords)
    xpad = jnp.zeros((N_POINTS, PAD_F), jnp.float32).at[:, :coords].set(xf)
    xt = xpad.T

    idx = _knn_indices(xpad, xt)
    idx_flat = idx.reshape(-1)

    g1 = _make_sc_gather(N_POINTS * K_NBRS, PAD_F, 512)(xpad, idx_flat)
    wt1 = jnp.zeros((PAD_F, 64), jnp.float32).at[:coords].set(W1a[:coords])
    wb1 = jnp.zeros((PAD_F, 64), jnp.float32).at[:coords].set(W1a[coords:])
    w1b_p = jnp.zeros((64, PAD_F), jnp.float32).at[:, :64].set(W1b)
    b1b_p = jnp.zeros((1, PAD_F), jnp.float32).at[:, :64].set(b1b)
    x1 = _edge_mlp(xpad, g1, wt1, wb1, b1a.reshape(1, -1),
                   w1b_p, b1b_p, relu_out=True)

    g2 = _make_sc_gather(N_POINTS * K_NBRS, PAD_F, 512)(x1, idx_flat)
    wt2 = jnp.zeros((PAD_F, 64), jnp.float32).at[:64].set(W2a[:64])
    wb2 = jnp.zeros((PAD_F, 64), jnp.float32).at[:64].set(W2a[64:])
    out = _edge_mlp(x1, g2, wt2, wb2, b2a.reshape(1, -1),
                    W2b, b2b.reshape(1, -1), relu_out=False)
    return out.reshape(batch, pts, -1)


# speedup vs baseline: 1.3250x; 1.3250x over previous
import functools

import jax
import jax.numpy as jnp
from jax import lax
from jax.experimental import pallas as pl
from jax.experimental.pallas import tpu as pltpu
from jax.experimental.pallas import tpu_sc as plsc

N_POINTS = 8192
K_NBRS = 8
KNN_ROWS = 256
NODE_TILE = 512
PAD_F = 128

SC_CORES = 2
SC_SUBCORES = 16
SC_WORKERS = SC_CORES * SC_SUBCORES


_SORT8 = [(0, 1), (2, 3), (4, 5), (6, 7),
          (0, 2), (1, 3), (4, 6), (5, 7),
          (1, 2), (5, 6),
          (0, 4), (1, 5), (2, 6), (3, 7),
          (2, 4), (3, 5),
          (1, 2), (3, 4), (5, 6)]
_BITONIC8 = [(0, 4), (1, 5), (2, 6), (3, 7),
             (0, 2), (1, 3), (4, 6), (5, 7),
             (0, 1), (2, 3), (4, 5), (6, 7)]


def _knn_body(xr_ref, xt_ref, out_ref):
    xr = xr_ref[...]
    xt = xt_ref[...]
    colsq = jnp.sum(xt * xt, axis=0, keepdims=True)
    cross = jax.lax.dot_general(
        xr, xt, (((1,), (0,)), ((), ())),
        preferred_element_type=jnp.float32)
    dist = colsq - 2.0 * cross

    i32 = jnp.int32
    keys0 = lax.bitcast_convert_type(dist, i32) & i32(-512)

    w = N_POINTS // K_NBRS
    vals = [lax.bitcast_convert_type(keys0[:, s * w:(s + 1) * w] | i32(s * 64),
                                     jnp.float32)
            for s in range(K_NBRS)]

    def ce(i, j):
        vi, vj = vals[i], vals[j]
        vals[i] = jnp.minimum(vi, vj)
        vals[j] = jnp.maximum(vi, vj)

    def bump(x, delta):
        return lax.bitcast_convert_type(
            lax.bitcast_convert_type(x, i32) + i32(delta), jnp.float32)

    for i, j in _SORT8:
        ce(i, j)
    while w > 16:
        w //= 2
        vals = [jnp.minimum(vals[i][:, :w], bump(vals[7 - i][:, w:], w // 16))
                for i in range(K_NBRS)]
        for i, j in _BITONIC8:
            ce(i, j)

    ck = jnp.concatenate(vals, axis=1)
    rows = ck.shape[0]
    pos16 = lax.broadcasted_iota(jnp.int32, (rows, 128), 1) & i32(15)
    inf = jnp.float32(jnp.inf)
    for k in range(K_NBRS):
        m = jnp.min(ck, axis=1, keepdims=True)
        p = jnp.min(jnp.where(ck == m, pos16, i32(16)), axis=1)
        track = lax.bitcast_convert_type(m, i32)[:, 0] & i32(511)
        out_ref[0, k, :] = track * 16 + p
        ck = jnp.where(ck == m, inf, ck)


def _knn_indices(xpad, xt):
    grid = N_POINTS // KNN_ROWS
    out = pl.pallas_call(
        _knn_body,
        grid=(grid,),
        in_specs=[
            pl.BlockSpec((KNN_ROWS, PAD_F), lambda i: (i, 0)),
            pl.BlockSpec((PAD_F, N_POINTS), lambda i: (0, 0)),
        ],
        out_specs=pl.BlockSpec((1, K_NBRS, KNN_ROWS), lambda i: (i, 0, 0)),
        out_shape=jax.ShapeDtypeStruct((grid, K_NBRS, KNN_ROWS), jnp.int32),
    )(xpad, xt)
    return jnp.transpose(out, (0, 2, 1)).reshape(N_POINTS, K_NBRS)


def _make_sc_gather(n_idx, feat, chunk):
    per_worker = n_idx // SC_WORKERS
    n_chunks = per_worker // chunk
    mesh = plsc.VectorSubcoreMesh(core_axis_name="c", subcore_axis_name="s")

    @functools.partial(
        pl.kernel,
        out_type=jax.ShapeDtypeStruct((n_idx, feat), jnp.float32),
        mesh=mesh,
        scratch_types=[
            pltpu.VMEM((chunk,), jnp.int32),
            pltpu.VMEM((chunk, feat), jnp.float32),
            pltpu.SemaphoreType.DMA,
        ],
    )
    def gather(table_hbm, idx_hbm, out_hbm, idx_v, rows_v, sem):
        wid = lax.axis_index("s") * SC_CORES + lax.axis_index("c")
        for c in range(n_chunks):
            base = wid * per_worker + c * chunk
            pltpu.sync_copy(idx_hbm.at[pl.ds(base, chunk)], idx_v)
            pltpu.async_copy(table_hbm.at[idx_v], rows_v, sem).wait()
            pltpu.sync_copy(rows_v, out_hbm.at[pl.ds(base, chunk)])

    return gather


def _mlp_body(x_ref, g_ref, wt_ref, wb_ref, ba_ref, w2_ref, b2_ref, o_ref,
              *, relu_out):
    xi = x_ref[...]
    t_nodes, feat = xi.shape
    n_edges = t_nodes * K_NBRS
    xj = g_ref[...].reshape(t_nodes, K_NBRS, feat)
    d = (xj - xi[:, None, :]).reshape(n_edges, feat)
    t1 = jax.lax.dot_general(xi, wt_ref[...], (((1,), (0,)), ((), ())),
                             preferred_element_type=jnp.float32)
    t2 = jax.lax.dot_general(d, wb_ref[...], (((1,), (0,)), ((), ())),
                             preferred_element_type=jnp.float32)
    hdim = t1.shape[1]
    pre = (t1[:, None, :] + t2.reshape(t_nodes, K_NBRS, hdim)
           + ba_ref[...][None])
    h = jnp.maximum(pre, 0.0).reshape(n_edges, hdim)
    h2 = jax.lax.dot_general(h, w2_ref[...], (((1,), (0,)), ((), ())),
                             preferred_element_type=jnp.float32) + b2_ref[...]
    odim = h2.shape[1]
    out = jnp.max(h2.reshape(t_nodes, K_NBRS, odim), axis=1)
    if relu_out:
        out = jnp.maximum(out, 0.0)
    o_ref[...] = out


def _edge_mlp(x, gathered, wt, wb, ba, w2, b2, relu_out):
    t = NODE_TILE
    grid = N_POINTS // t
    feat = x.shape[1]
    hdim = wt.shape[1]
    odim = w2.shape[1]
    body = functools.partial(_mlp_body, relu_out=relu_out)
    return pl.pallas_call(
        body,
        grid=(grid,),
        in_specs=[
            pl.BlockSpec((t, feat), lambda i: (i, 0)),
            pl.BlockSpec((t * K_NBRS, feat), lambda i: (i, 0)),
            pl.BlockSpec((feat, hdim), lambda i: (0, 0)),
            pl.BlockSpec((feat, hdim), lambda i: (0, 0)),
            pl.BlockSpec((1, hdim), lambda i: (0, 0)),
            pl.BlockSpec((hdim, odim), lambda i: (0, 0)),
            pl.BlockSpec((1, odim), lambda i: (0, 0)),
        ],
        out_specs=pl.BlockSpec((t, odim), lambda i: (i, 0)),
        out_shape=jax.ShapeDtypeStruct((N_POINTS, odim), jnp.float32),
    )(x, gathered, wt, wb, ba, w2, b2)


def kernel(point_cloud, W1a, b1a, W1b, b1b, W2a, b2a, W2b, b2b):
    batch, pts, coords = point_cloud.shape
    xf = point_cloud.reshape(-1, coords)
    xpad = jnp.zeros((N_POINTS, PAD_F), jnp.float32).at[:, :coords].set(xf)
    xt = xpad.T

    idx = _knn_indices(xpad, xt)
    idx_flat = idx.reshape(-1)

    g1 = _make_sc_gather(N_POINTS * K_NBRS, PAD_F, 512)(xpad, idx_flat)
    wt1 = jnp.zeros((PAD_F, 64), jnp.float32).at[:coords].set(W1a[:coords])
    wb1 = jnp.zeros((PAD_F, 64), jnp.float32).at[:coords].set(W1a[coords:])
    w1b_p = jnp.zeros((64, PAD_F), jnp.float32).at[:, :64].set(W1b)
    b1b_p = jnp.zeros((1, PAD_F), jnp.float32).at[:, :64].set(b1b)
    x1 = _edge_mlp(xpad, g1, wt1, wb1, b1a.reshape(1, -1),
                   w1b_p, b1b_p, relu_out=True)

    g2 = _make_sc_gather(N_POINTS * K_NBRS, PAD_F, 512)(x1, idx_flat)
    wt2 = jnp.zeros((PAD_F, 64), jnp.float32).at[:64].set(W2a[:64])
    wb2 = jnp.zeros((PAD_F, 64), jnp.float32).at[:64].set(W2a[64:])
    out = _edge_mlp(x1, g2, wt2, wb2, b2a.reshape(1, -1),
                    W2b, b2b.reshape(1, -1), relu_out=False)
    return out.reshape(batch, pts, -1)


# speedup vs baseline: 1.4067x; 1.0617x over previous
import functools

import jax
import jax.numpy as jnp
from jax import lax
from jax.experimental import pallas as pl
from jax.experimental.pallas import tpu as pltpu
from jax.experimental.pallas import tpu_sc as plsc

N_POINTS = 8192
K_NBRS = 8
KNN_ROWS = 256
NODE_TILE = 512
PAD_F = 128

SC_CORES = 2
SC_SUBCORES = 16
SC_WORKERS = SC_CORES * SC_SUBCORES


_SORT8 = [(0, 1), (2, 3), (4, 5), (6, 7),
          (0, 2), (1, 3), (4, 6), (5, 7),
          (1, 2), (5, 6),
          (0, 4), (1, 5), (2, 6), (3, 7),
          (2, 4), (3, 5),
          (1, 2), (3, 4), (5, 6)]
_BITONIC8 = [(0, 4), (1, 5), (2, 6), (3, 7),
             (0, 2), (1, 3), (4, 6), (5, 7),
             (0, 1), (2, 3), (4, 5), (6, 7)]


def _knn_body(xr_ref, xt_ref, out_ref):
    xr = xr_ref[...][:, :8]
    xt = xt_ref[...]
    colsq = jnp.sum(xt * xt, axis=0, keepdims=True)
    cross = jax.lax.dot_general(
        xr, xt, (((1,), (0,)), ((), ())),
        preferred_element_type=jnp.float32)
    dist = colsq - 2.0 * cross

    i32 = jnp.int32
    keys0 = lax.bitcast_convert_type(dist, i32) & i32(-512)

    w = N_POINTS // K_NBRS
    vals = [lax.bitcast_convert_type(keys0[:, s * w:(s + 1) * w] | i32(s * 64),
                                     jnp.float32)
            for s in range(K_NBRS)]

    def ce(i, j):
        vi, vj = vals[i], vals[j]
        vals[i] = jnp.minimum(vi, vj)
        vals[j] = jnp.maximum(vi, vj)

    def bump(x, delta):
        return lax.bitcast_convert_type(
            lax.bitcast_convert_type(x, i32) + i32(delta), jnp.float32)

    for i, j in _SORT8:
        ce(i, j)
    while w > 16:
        w //= 2
        vals = [jnp.minimum(vals[i][:, :w], bump(vals[7 - i][:, w:], w // 16))
                for i in range(K_NBRS)]
        for i, j in _BITONIC8:
            ce(i, j)

    ck = jnp.concatenate(vals, axis=1)
    rows = ck.shape[0]
    pos16 = lax.broadcasted_iota(jnp.int32, (rows, 128), 1) & i32(15)
    inf = jnp.float32(jnp.inf)
    for k in range(K_NBRS):
        m = jnp.min(ck, axis=1, keepdims=True)
        p = jnp.min(jnp.where(ck == m, pos16, i32(16)), axis=1)
        track = lax.bitcast_convert_type(m, i32)[:, 0] & i32(511)
        out_ref[0, k, :] = track * 16 + p
        ck = jnp.where(ck == m, inf, ck)


def _knn_indices(xpad, xt):
    grid = N_POINTS // KNN_ROWS
    out = pl.pallas_call(
        _knn_body,
        grid=(grid,),
        in_specs=[
            pl.BlockSpec((KNN_ROWS, PAD_F), lambda i: (i, 0)),
            pl.BlockSpec((8, N_POINTS), lambda i: (0, 0)),
        ],
        out_specs=pl.BlockSpec((1, K_NBRS, KNN_ROWS), lambda i: (i, 0, 0)),
        out_shape=jax.ShapeDtypeStruct((grid, K_NBRS, KNN_ROWS), jnp.int32),
    )(xpad, xt)
    return jnp.transpose(out, (0, 2, 1)).reshape(N_POINTS, K_NBRS)


def _make_sc_gather(n_idx, feat, chunk):
    per_worker = n_idx // SC_WORKERS
    n_chunks = per_worker // chunk
    mesh = plsc.VectorSubcoreMesh(core_axis_name="c", subcore_axis_name="s")

    @functools.partial(
        pl.kernel,
        out_type=jax.ShapeDtypeStruct((n_idx, feat), jnp.float32),
        mesh=mesh,
        scratch_types=[
            pltpu.VMEM((chunk,), jnp.int32),
            pltpu.VMEM((chunk, feat), jnp.float32),
            pltpu.SemaphoreType.DMA,
        ],
    )
    def gather(table_hbm, idx_hbm, out_hbm, idx_v, rows_v, sem):
        wid = lax.axis_index("s") * SC_CORES + lax.axis_index("c")
        for c in range(n_chunks):
            base = wid * per_worker + c * chunk
            pltpu.sync_copy(idx_hbm.at[pl.ds(base, chunk)], idx_v)
            pltpu.async_copy(table_hbm.at[idx_v], rows_v, sem).wait()
            pltpu.sync_copy(rows_v, out_hbm.at[pl.ds(base, chunk)])

    return gather


def _mlp_body(x_ref, g_ref, wt_ref, wb_ref, ba_ref, w2_ref, b2_ref, o_ref,
              *, relu_out):
    xi = x_ref[...]
    t_nodes, feat = xi.shape
    n_edges = t_nodes * K_NBRS
    xj = g_ref[...].reshape(t_nodes, K_NBRS, feat)
    d = (xj - xi[:, None, :]).reshape(n_edges, feat)
    t1 = jax.lax.dot_general(xi, wt_ref[...], (((1,), (0,)), ((), ())),
                             preferred_element_type=jnp.float32)
    t2 = jax.lax.dot_general(d, wb_ref[...], (((1,), (0,)), ((), ())),
                             preferred_element_type=jnp.float32)
    hdim = t1.shape[1]
    pre = (t1[:, None, :] + t2.reshape(t_nodes, K_NBRS, hdim)
           + ba_ref[...][None])
    h = jnp.maximum(pre, 0.0).reshape(n_edges, hdim)
    h2 = jax.lax.dot_general(h, w2_ref[...], (((1,), (0,)), ((), ())),
                             preferred_element_type=jnp.float32) + b2_ref[...]
    odim = h2.shape[1]
    out = jnp.max(h2.reshape(t_nodes, K_NBRS, odim), axis=1)
    if relu_out:
        out = jnp.maximum(out, 0.0)
    o_ref[...] = out


def _edge_mlp(x, gathered, wt, wb, ba, w2, b2, relu_out):
    t = NODE_TILE
    grid = N_POINTS // t
    feat = x.shape[1]
    hdim = wt.shape[1]
    odim = w2.shape[1]
    body = functools.partial(_mlp_body, relu_out=relu_out)
    return pl.pallas_call(
        body,
        grid=(grid,),
        in_specs=[
            pl.BlockSpec((t, feat), lambda i: (i, 0)),
            pl.BlockSpec((t * K_NBRS, feat), lambda i: (i, 0)),
            pl.BlockSpec((feat, hdim), lambda i: (0, 0)),
            pl.BlockSpec((feat, hdim), lambda i: (0, 0)),
            pl.BlockSpec((1, hdim), lambda i: (0, 0)),
            pl.BlockSpec((hdim, odim), lambda i: (0, 0)),
            pl.BlockSpec((1, odim), lambda i: (0, 0)),
        ],
        out_specs=pl.BlockSpec((t, odim), lambda i: (i, 0)),
        out_shape=jax.ShapeDtypeStruct((N_POINTS, odim), jnp.float32),
    )(x, gathered, wt, wb, ba, w2, b2)


def kernel(point_cloud, W1a, b1a, W1b, b1b, W2a, b2a, W2b, b2b):
    batch, pts, coords = point_cloud.shape
    xf = point_cloud.reshape(-1, coords)
    xpad = jnp.zeros((N_POINTS, PAD_F), jnp.float32).at[:, :coords].set(xf)
    xt = jnp.zeros((8, N_POINTS), jnp.float32).at[:coords].set(xf.T)

    idx = _knn_indices(xpad, xt)
    idx_flat = idx.reshape(-1)

    g1 = _make_sc_gather(N_POINTS * K_NBRS, PAD_F, 512)(xpad, idx_flat)
    wt1 = jnp.zeros((PAD_F, 64), jnp.float32).at[:coords].set(W1a[:coords])
    wb1 = jnp.zeros((PAD_F, 64), jnp.float32).at[:coords].set(W1a[coords:])
    w1b_p = jnp.zeros((64, PAD_F), jnp.float32).at[:, :64].set(W1b)
    b1b_p = jnp.zeros((1, PAD_F), jnp.float32).at[:, :64].set(b1b)
    x1 = _edge_mlp(xpad, g1, wt1, wb1, b1a.reshape(1, -1),
                   w1b_p, b1b_p, relu_out=True)

    g2 = _make_sc_gather(N_POINTS * K_NBRS, PAD_F, 512)(x1, idx_flat)
    wt2 = jnp.zeros((PAD_F, 64), jnp.float32).at[:64].set(W2a[:64])
    wb2 = jnp.zeros((PAD_F, 64), jnp.float32).at[:64].set(W2a[64:])
    out = _edge_mlp(x1, g2, wt2, wb2, b2a.reshape(1, -1),
                    W2b, b2b.reshape(1, -1), relu_out=False)
    return out.reshape(batch, pts, -1)


# speedup vs baseline: 1.5978x; 1.1359x over previous
import functools

import jax
import jax.numpy as jnp
from jax import lax
from jax.experimental import pallas as pl
from jax.experimental.pallas import tpu as pltpu
from jax.experimental.pallas import tpu_sc as plsc

N_POINTS = 8192
K_NBRS = 8
KNN_ROWS = 512
NODE_TILE = 512
PAD_F = 128

SC_CORES = 2
SC_SUBCORES = 16
SC_WORKERS = SC_CORES * SC_SUBCORES


_SORT8 = [(0, 1), (2, 3), (4, 5), (6, 7),
          (0, 2), (1, 3), (4, 6), (5, 7),
          (1, 2), (5, 6),
          (0, 4), (1, 5), (2, 6), (3, 7),
          (2, 4), (3, 5),
          (1, 2), (3, 4), (5, 6)]
_BITONIC8 = [(0, 4), (1, 5), (2, 6), (3, 7),
             (0, 2), (1, 3), (4, 6), (5, 7),
             (0, 1), (2, 3), (4, 5), (6, 7)]


def _knn_body(xr_ref, xt_ref, out_ref):
    xr = xr_ref[...][:, :8]
    xt = xt_ref[...]
    colsq = jnp.sum(xt * xt, axis=0, keepdims=True)
    cross = jax.lax.dot_general(
        xr, xt, (((1,), (0,)), ((), ())),
        preferred_element_type=jnp.float32)
    dist = colsq - 2.0 * cross

    i32 = jnp.int32
    keys0 = lax.bitcast_convert_type(dist, i32) & i32(-512)

    w = N_POINTS // K_NBRS
    vals = [lax.bitcast_convert_type(keys0[:, s * w:(s + 1) * w] | i32(s * 64),
                                     jnp.float32)
            for s in range(K_NBRS)]

    def ce(i, j):
        vi, vj = vals[i], vals[j]
        vals[i] = jnp.minimum(vi, vj)
        vals[j] = jnp.maximum(vi, vj)

    def bump(x, delta):
        return lax.bitcast_convert_type(
            lax.bitcast_convert_type(x, i32) + i32(delta), jnp.float32)

    for i, j in _SORT8:
        ce(i, j)
    while w > 16:
        w //= 2
        vals = [jnp.minimum(vals[i][:, :w], bump(vals[7 - i][:, w:], w // 16))
                for i in range(K_NBRS)]
        for i, j in _BITONIC8:
            ce(i, j)

    ck = jnp.concatenate(vals, axis=1)
    rows = ck.shape[0]
    pos16 = lax.broadcasted_iota(jnp.int32, (rows, 128), 1) & i32(15)
    inf = jnp.float32(jnp.inf)
    for k in range(K_NBRS):
        m = jnp.min(ck, axis=1, keepdims=True)
        p = jnp.min(jnp.where(ck == m, pos16, i32(16)), axis=1)
        track = lax.bitcast_convert_type(m, i32)[:, 0] & i32(511)
        out_ref[0, k, :] = track * 16 + p
        ck = jnp.where(ck == m, inf, ck)


def _knn_indices(xpad, xt):
    grid = N_POINTS // KNN_ROWS
    out = pl.pallas_call(
        _knn_body,
        grid=(grid,),
        in_specs=[
            pl.BlockSpec((KNN_ROWS, PAD_F), lambda i: (i, 0)),
            pl.BlockSpec((8, N_POINTS), lambda i: (0, 0)),
        ],
        out_specs=pl.BlockSpec((1, K_NBRS, KNN_ROWS), lambda i: (i, 0, 0)),
        out_shape=jax.ShapeDtypeStruct((grid, K_NBRS, KNN_ROWS), jnp.int32),
    )(xpad, xt)
    return jnp.transpose(out, (0, 2, 1)).reshape(N_POINTS, K_NBRS)


def _make_sc_gather(n_idx, feat, chunk):
    per_worker = n_idx // SC_WORKERS
    n_chunks = per_worker // chunk
    mesh = plsc.VectorSubcoreMesh(core_axis_name="c", subcore_axis_name="s")

    @functools.partial(
        pl.kernel,
        out_type=jax.ShapeDtypeStruct((n_idx, feat), jnp.float32),
        mesh=mesh,
        scratch_types=[
            pltpu.VMEM((chunk,), jnp.int32),
            pltpu.VMEM((chunk, feat), jnp.float32),
            pltpu.SemaphoreType.DMA,
        ],
    )
    def gather(table_hbm, idx_hbm, out_hbm, idx_v, rows_v, sem):
        wid = lax.axis_index("s") * SC_CORES + lax.axis_index("c")
        for c in range(n_chunks):
            base = wid * per_worker + c * chunk
            pltpu.sync_copy(idx_hbm.at[pl.ds(base, chunk)], idx_v)
            pltpu.async_copy(table_hbm.at[idx_v], rows_v, sem).wait()
            pltpu.sync_copy(rows_v, out_hbm.at[pl.ds(base, chunk)])

    return gather


def _mlp_body(x_ref, g_ref, wt_ref, wb_ref, ba_ref, w2_ref, b2_ref, o_ref,
              *, relu_out):
    xi = x_ref[...]
    t_nodes, feat = xi.shape
    n_edges = t_nodes * K_NBRS
    xj = g_ref[...].reshape(t_nodes, K_NBRS, feat)
    d = (xj - xi[:, None, :]).reshape(n_edges, feat)
    t1 = jax.lax.dot_general(xi, wt_ref[...], (((1,), (0,)), ((), ())),
                             preferred_element_type=jnp.float32)
    t2 = jax.lax.dot_general(d, wb_ref[...], (((1,), (0,)), ((), ())),
                             preferred_element_type=jnp.float32)
    hdim = t1.shape[1]
    pre = (t1[:, None, :] + t2.reshape(t_nodes, K_NBRS, hdim)
           + ba_ref[...][None])
    h = jnp.maximum(pre, 0.0).reshape(n_edges, hdim)
    h2 = jax.lax.dot_general(h, w2_ref[...], (((1,), (0,)), ((), ())),
                             preferred_element_type=jnp.float32) + b2_ref[...]
    odim = h2.shape[1]
    out = jnp.max(h2.reshape(t_nodes, K_NBRS, odim), axis=1)
    if relu_out:
        out = jnp.maximum(out, 0.0)
    o_ref[...] = out


def _edge_mlp(x, gathered, wt, wb, ba, w2, b2, relu_out):
    t = NODE_TILE
    grid = N_POINTS // t
    feat = x.shape[1]
    hdim = wt.shape[1]
    odim = w2.shape[1]
    body = functools.partial(_mlp_body, relu_out=relu_out)
    return pl.pallas_call(
        body,
        grid=(grid,),
        in_specs=[
            pl.BlockSpec((t, feat), lambda i: (i, 0)),
            pl.BlockSpec((t * K_NBRS, feat), lambda i: (i, 0)),
            pl.BlockSpec((feat, hdim), lambda i: (0, 0)),
            pl.BlockSpec((feat, hdim), lambda i: (0, 0)),
            pl.BlockSpec((1, hdim), lambda i: (0, 0)),
            pl.BlockSpec((hdim, odim), lambda i: (0, 0)),
            pl.BlockSpec((1, odim), lambda i: (0, 0)),
        ],
        out_specs=pl.BlockSpec((t, odim), lambda i: (i, 0)),
        out_shape=jax.ShapeDtypeStruct((N_POINTS, odim), jnp.float32),
    )(x, gathered, wt, wb, ba, w2, b2)


def kernel(point_cloud, W1a, b1a, W1b, b1b, W2a, b2a, W2b, b2b):
    batch, pts, coords = point_cloud.shape
    xf = point_cloud.reshape(-1, coords)
    xpad = jnp.zeros((N_POINTS, PAD_F), jnp.float32).at[:, :coords].set(xf)
    xt = jnp.zeros((8, N_POINTS), jnp.float32).at[:coords].set(xf.T)

    idx = _knn_indices(xpad, xt)
    idx_flat = idx.reshape(-1)

    g1 = _make_sc_gather(N_POINTS * K_NBRS, PAD_F, 512)(xpad, idx_flat)
    wt1 = jnp.zeros((PAD_F, 64), jnp.float32).at[:coords].set(W1a[:coords])
    wb1 = jnp.zeros((PAD_F, 64), jnp.float32).at[:coords].set(W1a[coords:])
    w1b_p = jnp.zeros((64, PAD_F), jnp.float32).at[:, :64].set(W1b)
    b1b_p = jnp.zeros((1, PAD_F), jnp.float32).at[:, :64].set(b1b)
    x1 = _edge_mlp(xpad, g1, wt1, wb1, b1a.reshape(1, -1),
                   w1b_p, b1b_p, relu_out=True)

    g2 = _make_sc_gather(N_POINTS * K_NBRS, PAD_F, 512)(x1, idx_flat)
    wt2 = jnp.zeros((PAD_F, 64), jnp.float32).at[:64].set(W2a[:64])
    wb2 = jnp.zeros((PAD_F, 64), jnp.float32).at[:64].set(W2a[64:])
    out = _edge_mlp(x1, g2, wt2, wb2, b2a.reshape(1, -1),
                    W2b, b2b.reshape(1, -1), relu_out=False)
    return out.reshape(batch, pts, -1)


# speedup vs baseline: 1.6068x; 1.0056x over previous
import functools

import jax
import jax.numpy as jnp
from jax import lax
from jax.experimental import pallas as pl
from jax.experimental.pallas import tpu as pltpu
from jax.experimental.pallas import tpu_sc as plsc

N_POINTS = 8192
K_NBRS = 8
KNN_ROWS = 1024
NODE_TILE = 512
PAD_F = 128

SC_CORES = 2
SC_SUBCORES = 16
SC_WORKERS = SC_CORES * SC_SUBCORES


_SORT8 = [(0, 1), (2, 3), (4, 5), (6, 7),
          (0, 2), (1, 3), (4, 6), (5, 7),
          (1, 2), (5, 6),
          (0, 4), (1, 5), (2, 6), (3, 7),
          (2, 4), (3, 5),
          (1, 2), (3, 4), (5, 6)]
_BITONIC8 = [(0, 4), (1, 5), (2, 6), (3, 7),
             (0, 2), (1, 3), (4, 6), (5, 7),
             (0, 1), (2, 3), (4, 5), (6, 7)]


def _knn_body(xr_ref, xt_ref, out_ref):
    xr = xr_ref[...][:, :8]
    xt = xt_ref[...]
    colsq = jnp.sum(xt * xt, axis=0, keepdims=True)
    cross = jax.lax.dot_general(
        xr, xt, (((1,), (0,)), ((), ())),
        preferred_element_type=jnp.float32)
    dist = colsq - 2.0 * cross

    i32 = jnp.int32
    keys0 = lax.bitcast_convert_type(dist, i32) & i32(-512)

    w = N_POINTS // K_NBRS
    vals = [lax.bitcast_convert_type(keys0[:, s * w:(s + 1) * w] | i32(s * 64),
                                     jnp.float32)
            for s in range(K_NBRS)]

    def ce(i, j):
        vi, vj = vals[i], vals[j]
        vals[i] = jnp.minimum(vi, vj)
        vals[j] = jnp.maximum(vi, vj)

    def bump(x, delta):
        return lax.bitcast_convert_type(
            lax.bitcast_convert_type(x, i32) + i32(delta), jnp.float32)

    for i, j in _SORT8:
        ce(i, j)
    while w > 16:
        w //= 2
        vals = [jnp.minimum(vals[i][:, :w], bump(vals[7 - i][:, w:], w // 16))
                for i in range(K_NBRS)]
        for i, j in _BITONIC8:
            ce(i, j)

    ck = jnp.concatenate(vals, axis=1)
    rows = ck.shape[0]
    pos16 = lax.broadcasted_iota(jnp.int32, (rows, 128), 1) & i32(15)
    inf = jnp.float32(jnp.inf)
    for k in range(K_NBRS):
        m = jnp.min(ck, axis=1, keepdims=True)
        p = jnp.min(jnp.where(ck == m, pos16, i32(16)), axis=1)
        track = lax.bitcast_convert_type(m, i32)[:, 0] & i32(511)
        out_ref[0, k, :] = track * 16 + p
        ck = jnp.where(ck == m, inf, ck)


def _knn_indices(xpad, xt):
    grid = N_POINTS // KNN_ROWS
    out = pl.pallas_call(
        _knn_body,
        grid=(grid,),
        in_specs=[
            pl.BlockSpec((KNN_ROWS, PAD_F), lambda i: (i, 0)),
            pl.BlockSpec((8, N_POINTS), lambda i: (0, 0)),
        ],
        out_specs=pl.BlockSpec((1, K_NBRS, KNN_ROWS), lambda i: (i, 0, 0)),
        out_shape=jax.ShapeDtypeStruct((grid, K_NBRS, KNN_ROWS), jnp.int32),
    )(xpad, xt)
    return jnp.transpose(out, (0, 2, 1)).reshape(N_POINTS, K_NBRS)


def _make_sc_gather(n_idx, feat, chunk):
    per_worker = n_idx // SC_WORKERS
    n_chunks = per_worker // chunk
    mesh = plsc.VectorSubcoreMesh(core_axis_name="c", subcore_axis_name="s")

    @functools.partial(
        pl.kernel,
        out_type=jax.ShapeDtypeStruct((n_idx, feat), jnp.float32),
        mesh=mesh,
        scratch_types=[
            pltpu.VMEM((chunk,), jnp.int32),
            pltpu.VMEM((chunk, feat), jnp.float32),
            pltpu.SemaphoreType.DMA,
        ],
    )
    def gather(table_hbm, idx_hbm, out_hbm, idx_v, rows_v, sem):
        wid = lax.axis_index("s") * SC_CORES + lax.axis_index("c")
        for c in range(n_chunks):
            base = wid * per_worker + c * chunk
            pltpu.sync_copy(idx_hbm.at[pl.ds(base, chunk)], idx_v)
            pltpu.async_copy(table_hbm.at[idx_v], rows_v, sem).wait()
            pltpu.sync_copy(rows_v, out_hbm.at[pl.ds(base, chunk)])

    return gather


def _mlp_body(x_ref, g_ref, wt_ref, wb_ref, ba_ref, w2_ref, b2_ref, o_ref,
              *, relu_out):
    xi = x_ref[...]
    t_nodes, feat = xi.shape
    n_edges = t_nodes * K_NBRS
    xj = g_ref[...].reshape(t_nodes, K_NBRS, feat)
    d = (xj - xi[:, None, :]).reshape(n_edges, feat)
    t1 = jax.lax.dot_general(xi, wt_ref[...], (((1,), (0,)), ((), ())),
                             preferred_element_type=jnp.float32)
    t2 = jax.lax.dot_general(d, wb_ref[...], (((1,), (0,)), ((), ())),
                             preferred_element_type=jnp.float32)
    hdim = t1.shape[1]
    pre = (t1[:, None, :] + t2.reshape(t_nodes, K_NBRS, hdim)
           + ba_ref[...][None])
    h = jnp.maximum(pre, 0.0).reshape(n_edges, hdim)
    h2 = jax.lax.dot_general(h, w2_ref[...], (((1,), (0,)), ((), ())),
                             preferred_element_type=jnp.float32) + b2_ref[...]
    odim = h2.shape[1]
    out = jnp.max(h2.reshape(t_nodes, K_NBRS, odim), axis=1)
    if relu_out:
        out = jnp.maximum(out, 0.0)
    o_ref[...] = out


def _edge_mlp(x, gathered, wt, wb, ba, w2, b2, relu_out):
    t = NODE_TILE
    grid = N_POINTS // t
    feat = x.shape[1]
    hdim = wt.shape[1]
    odim = w2.shape[1]
    body = functools.partial(_mlp_body, relu_out=relu_out)
    return pl.pallas_call(
        body,
        grid=(grid,),
        in_specs=[
            pl.BlockSpec((t, feat), lambda i: (i, 0)),
            pl.BlockSpec((t * K_NBRS, feat), lambda i: (i, 0)),
            pl.BlockSpec((feat, hdim), lambda i: (0, 0)),
            pl.BlockSpec((feat, hdim), lambda i: (0, 0)),
            pl.BlockSpec((1, hdim), lambda i: (0, 0)),
            pl.BlockSpec((hdim, odim), lambda i: (0, 0)),
            pl.BlockSpec((1, odim), lambda i: (0, 0)),
        ],
        out_specs=pl.BlockSpec((t, odim), lambda i: (i, 0)),
        out_shape=jax.ShapeDtypeStruct((N_POINTS, odim), jnp.float32),
    )(x, gathered, wt, wb, ba, w2, b2)


def kernel(point_cloud, W1a, b1a, W1b, b1b, W2a, b2a, W2b, b2b):
    batch, pts, coords = point_cloud.shape
    xf = point_cloud.reshape(-1, coords)
    xpad = jnp.zeros((N_POINTS, PAD_F), jnp.float32).at[:, :coords].set(xf)
    xt = jnp.zeros((8, N_POINTS), jnp.float32).at[:coords].set(xf.T)

    idx = _knn_indices(xpad, xt)
    idx_flat = idx.reshape(-1)

    g1 = _make_sc_gather(N_POINTS * K_NBRS, PAD_F, 512)(xpad, idx_flat)
    wt1 = jnp.zeros((PAD_F, 64), jnp.float32).at[:coords].set(W1a[:coords])
    wb1 = jnp.zeros((PAD_F, 64), jnp.float32).at[:coords].set(W1a[coords:])
    w1b_p = jnp.zeros((64, PAD_F), jnp.float32).at[:, :64].set(W1b)
    b1b_p = jnp.zeros((1, PAD_F), jnp.float32).at[:, :64].set(b1b)
    x1 = _edge_mlp(xpad, g1, wt1, wb1, b1a.reshape(1, -1),
                   w1b_p, b1b_p, relu_out=True)

    g2 = _make_sc_gather(N_POINTS * K_NBRS, PAD_F, 512)(x1, idx_flat)
    wt2 = jnp.zeros((PAD_F, 64), jnp.float32).at[:64].set(W2a[:64])
    wb2 = jnp.zeros((PAD_F, 64), jnp.float32).at[:64].set(W2a[64:])
    out = _edge_mlp(x1, g2, wt2, wb2, b2a.reshape(1, -1),
                    W2b, b2b.reshape(1, -1), relu_out=False)
    return out.reshape(batch, pts, -1)


# speedup vs baseline: 1.6171x; 1.0064x over previous
import functools

import jax
import jax.numpy as jnp
from jax import lax
from jax.experimental import pallas as pl
from jax.experimental.pallas import tpu as pltpu
from jax.experimental.pallas import tpu_sc as plsc

N_POINTS = 8192
K_NBRS = 8
KNN_ROWS = 1024
NODE_TILE = 1024
PAD_F = 128

SC_CORES = 2
SC_SUBCORES = 16
SC_WORKERS = SC_CORES * SC_SUBCORES


_SORT8 = [(0, 1), (2, 3), (4, 5), (6, 7),
          (0, 2), (1, 3), (4, 6), (5, 7),
          (1, 2), (5, 6),
          (0, 4), (1, 5), (2, 6), (3, 7),
          (2, 4), (3, 5),
          (1, 2), (3, 4), (5, 6)]
_BITONIC8 = [(0, 4), (1, 5), (2, 6), (3, 7),
             (0, 2), (1, 3), (4, 6), (5, 7),
             (0, 1), (2, 3), (4, 5), (6, 7)]


def _knn_body(xr_ref, xt_ref, out_ref):
    xr = xr_ref[...][:, :8]
    xt = xt_ref[...]
    colsq = jnp.sum(xt * xt, axis=0, keepdims=True)
    cross = jax.lax.dot_general(
        xr, xt, (((1,), (0,)), ((), ())),
        preferred_element_type=jnp.float32)
    dist = colsq - 2.0 * cross

    i32 = jnp.int32
    keys0 = lax.bitcast_convert_type(dist, i32) & i32(-512)

    w = N_POINTS // K_NBRS
    vals = [lax.bitcast_convert_type(keys0[:, s * w:(s + 1) * w] | i32(s * 64),
                                     jnp.float32)
            for s in range(K_NBRS)]

    def ce(i, j):
        vi, vj = vals[i], vals[j]
        vals[i] = jnp.minimum(vi, vj)
        vals[j] = jnp.maximum(vi, vj)

    def bump(x, delta):
        return lax.bitcast_convert_type(
            lax.bitcast_convert_type(x, i32) + i32(delta), jnp.float32)

    for i, j in _SORT8:
        ce(i, j)
    while w > 16:
        w //= 2
        vals = [jnp.minimum(vals[i][:, :w], bump(vals[7 - i][:, w:], w // 16))
                for i in range(K_NBRS)]
        for i, j in _BITONIC8:
            ce(i, j)

    ck = jnp.concatenate(vals, axis=1)
    rows = ck.shape[0]
    pos16 = lax.broadcasted_iota(jnp.int32, (rows, 128), 1) & i32(15)
    inf = jnp.float32(jnp.inf)
    for k in range(K_NBRS):
        m = jnp.min(ck, axis=1, keepdims=True)
        p = jnp.min(jnp.where(ck == m, pos16, i32(16)), axis=1)
        track = lax.bitcast_convert_type(m, i32)[:, 0] & i32(511)
        out_ref[0, k, :] = track * 16 + p
        ck = jnp.where(ck == m, inf, ck)


def _knn_indices(xpad, xt):
    grid = N_POINTS // KNN_ROWS
    out = pl.pallas_call(
        _knn_body,
        grid=(grid,),
        in_specs=[
            pl.BlockSpec((KNN_ROWS, PAD_F), lambda i: (i, 0)),
            pl.BlockSpec((8, N_POINTS), lambda i: (0, 0)),
        ],
        out_specs=pl.BlockSpec((1, K_NBRS, KNN_ROWS), lambda i: (i, 0, 0)),
        out_shape=jax.ShapeDtypeStruct((grid, K_NBRS, KNN_ROWS), jnp.int32),
    )(xpad, xt)
    return jnp.transpose(out, (0, 2, 1)).reshape(N_POINTS, K_NBRS)


def _make_sc_gather(n_idx, feat, chunk):
    per_worker = n_idx // SC_WORKERS
    n_chunks = per_worker // chunk
    mesh = plsc.VectorSubcoreMesh(core_axis_name="c", subcore_axis_name="s")

    @functools.partial(
        pl.kernel,
        out_type=jax.ShapeDtypeStruct((n_idx, feat), jnp.float32),
        mesh=mesh,
        scratch_types=[
            pltpu.VMEM((chunk,), jnp.int32),
            pltpu.VMEM((chunk,), jnp.int32),
            pltpu.VMEM((chunk, feat), jnp.float32),
            pltpu.VMEM((chunk, feat), jnp.float32),
            pltpu.SemaphoreType.DMA,
            pltpu.SemaphoreType.DMA,
        ],
    )
    def gather(table_hbm, idx_hbm, out_hbm, iv0, iv1, rv0, rv1, s0, s1):
        wid = lax.axis_index("s") * SC_CORES + lax.axis_index("c")
        base = wid * per_worker
        ivs, rvs, sems = [iv0, iv1], [rv0, rv1], [s0, s1]
        pltpu.sync_copy(idx_hbm.at[pl.ds(base, chunk)], ivs[0])
        pending = [pltpu.async_copy(table_hbm.at[ivs[0]], rvs[0], sems[0])]
        for c in range(n_chunks):
            b = c % 2
            if c + 1 < n_chunks:
                nb = (c + 1) % 2
                pltpu.sync_copy(
                    idx_hbm.at[pl.ds(base + (c + 1) * chunk, chunk)], ivs[nb])
                pending.append(
                    pltpu.async_copy(table_hbm.at[ivs[nb]], rvs[nb], sems[nb]))
            pending[c].wait()
            pltpu.sync_copy(rvs[b], out_hbm.at[pl.ds(base + c * chunk, chunk)])

    return gather


def _mlp_body(x_ref, g_ref, wt_ref, wb_ref, ba_ref, w2_ref, b2_ref, o_ref,
              *, relu_out):
    xi = x_ref[...]
    t_nodes, feat = xi.shape
    n_edges = t_nodes * K_NBRS
    xj = g_ref[...].reshape(t_nodes, K_NBRS, feat)
    d = (xj - xi[:, None, :]).reshape(n_edges, feat)
    t1 = jax.lax.dot_general(xi, wt_ref[...], (((1,), (0,)), ((), ())),
                             preferred_element_type=jnp.float32)
    t2 = jax.lax.dot_general(d, wb_ref[...], (((1,), (0,)), ((), ())),
                             preferred_element_type=jnp.float32)
    hdim = t1.shape[1]
    pre = (t1[:, None, :] + t2.reshape(t_nodes, K_NBRS, hdim)
           + ba_ref[...][None])
    h = jnp.maximum(pre, 0.0).reshape(n_edges, hdim)
    h2 = jax.lax.dot_general(h, w2_ref[...], (((1,), (0,)), ((), ())),
                             preferred_element_type=jnp.float32) + b2_ref[...]
    odim = h2.shape[1]
    out = jnp.max(h2.reshape(t_nodes, K_NBRS, odim), axis=1)
    if relu_out:
        out = jnp.maximum(out, 0.0)
    o_ref[...] = out


def _edge_mlp(x, gathered, wt, wb, ba, w2, b2, relu_out):
    t = NODE_TILE
    grid = N_POINTS // t
    feat = x.shape[1]
    hdim = wt.shape[1]
    odim = w2.shape[1]
    body = functools.partial(_mlp_body, relu_out=relu_out)
    return pl.pallas_call(
        body,
        grid=(grid,),
        in_specs=[
            pl.BlockSpec((t, feat), lambda i: (i, 0)),
            pl.BlockSpec((t * K_NBRS, feat), lambda i: (i, 0)),
            pl.BlockSpec((feat, hdim), lambda i: (0, 0)),
            pl.BlockSpec((feat, hdim), lambda i: (0, 0)),
            pl.BlockSpec((1, hdim), lambda i: (0, 0)),
            pl.BlockSpec((hdim, odim), lambda i: (0, 0)),
            pl.BlockSpec((1, odim), lambda i: (0, 0)),
        ],
        out_specs=pl.BlockSpec((t, odim), lambda i: (i, 0)),
        out_shape=jax.ShapeDtypeStruct((N_POINTS, odim), jnp.float32),
    )(x, gathered, wt, wb, ba, w2, b2)


def kernel(point_cloud, W1a, b1a, W1b, b1b, W2a, b2a, W2b, b2b):
    batch, pts, coords = point_cloud.shape
    xf = point_cloud.reshape(-1, coords)
    xpad = jnp.zeros((N_POINTS, PAD_F), jnp.float32).at[:, :coords].set(xf)
    xt = jnp.zeros((8, N_POINTS), jnp.float32).at[:coords].set(xf.T)

    idx = _knn_indices(xpad, xt)
    idx_flat = idx.reshape(-1)

    g1 = _make_sc_gather(N_POINTS * K_NBRS, PAD_F, 256)(xpad, idx_flat)
    wt1 = jnp.zeros((PAD_F, 64), jnp.float32).at[:coords].set(W1a[:coords])
    wb1 = jnp.zeros((PAD_F, 64), jnp.float32).at[:coords].set(W1a[coords:])
    w1b_p = jnp.zeros((64, PAD_F), jnp.float32).at[:, :64].set(W1b)
    b1b_p = jnp.zeros((1, PAD_F), jnp.float32).at[:, :64].set(b1b)
    x1 = _edge_mlp(xpad, g1, wt1, wb1, b1a.reshape(1, -1),
                   w1b_p, b1b_p, relu_out=True)

    g2 = _make_sc_gather(N_POINTS * K_NBRS, PAD_F, 256)(x1, idx_flat)
    wt2 = jnp.zeros((PAD_F, 64), jnp.float32).at[:64].set(W2a[:64])
    wb2 = jnp.zeros((PAD_F, 64), jnp.float32).at[:64].set(W2a[64:])
    out = _edge_mlp(x1, g2, wt2, wb2, b2a.reshape(1, -1),
                    W2b, b2b.reshape(1, -1), relu_out=False)
    return out.reshape(batch, pts, -1)
